# baseline (speedup 1.0000x reference)
import functools

import jax
import jax.numpy as jnp
from jax import lax
from jax.experimental import pallas as pl
from jax.experimental.pallas import tpu as pltpu
from jax.experimental.pallas import tpu_sc as plsc

H, W, C = 64, 64, 256
PH, PW = 7, 7
N = 1000

NC, NS = 2, 16
NW = NC * NS
NSEG = PH * PW
CHUNK = 128
FULL_CPS = N // CHUNK
A_UNITS = NSEG * FULL_CPS
TAIL_START = FULL_CPS * CHUNK
TAIL = N - TAIL_START
A_PIPE = 10
TOTAL_ROWS = NSEG * N
NPAD = 1024


def _seg_indices(props_v, idx_v, par, s, p0, ngroups):
    i = (s * 9363) >> 16
    j = s - i * PW
    for g in range(ngroups):
        off = p0 + g * 16
        x0 = props_v[0, pl.ds(off, 16)]
        y0 = props_v[1, pl.ds(off, 16)]
        x1 = props_v[2, pl.ds(off, 16)]
        y1 = props_v[3, pl.ds(off, 16)]
        xmin = x0.astype(jnp.int32)
        ymin = y0.astype(jnp.int32)
        xt = x1.astype(jnp.int32)
        yt = y1.astype(jnp.int32)
        xmax = jnp.where(x1 > xt.astype(jnp.float32), xt + 1, xt)
        ymax = jnp.where(y1 > yt.astype(jnp.float32), yt + 1, yt)
        hh = jnp.maximum(ymax - ymin, 1)
        ww = jnp.maximum(xmax - xmin, 1)
        r = jnp.minimum(((2 * i + 1) * hh * 4682) >> 16, hh - 1) + ymin
        r = jnp.clip(r, 0, H - 1)
        c = jnp.minimum(((2 * j + 1) * ww * 4682) >> 16, ww - 1) + xmin
        c = jnp.clip(c, 0, W - 1)
        idx_v[par, pl.ds(g * 16, 16)] = r * W + c


def _a_unit(props_v, idx_v, u, par):
    q = (u * 1338) >> 16
    s = u - q * NSEG
    p0 = q * CHUNK
    _seg_indices(props_v, idx_v, par, s, p0, CHUNK // 16)
    return s * N + p0


def _b_unit(props_v, idx_v, s, par):
    _seg_indices(props_v, idx_v, par, s, TAIL_START, 7)
    return s * N + TAIL_START


def _body(fm, props, out, props_v, idx_v, gbuf, gsem, wsem):
    wid = lax.axis_index("s") * NC + lax.axis_index("c")
    pltpu.sync_copy(props, props_v)

    NU = A_PIPE + 1
    gets = [None] * NU
    puts = [None] * NU
    rows = [None] * NU
    sizes = [CHUNK] * A_PIPE + [TAIL]
    for t in range(NU):
        par = t % 3
        if t >= 3:
            puts[t - 3].wait()
        if t < A_PIPE:
            rows[t] = _a_unit(props_v, idx_v, wid + NW * t, t % 2)
        else:
            rows[t] = _b_unit(props_v, idx_v, wid, t % 2)
        gets[t] = pltpu.async_copy(
            fm.at[idx_v.at[t % 2, pl.ds(0, sizes[t])]],
            gbuf.at[par, pl.ds(0, sizes[t])], gsem.at[par])
        if t > 0:
            gets[t - 1].wait()
            puts[t - 1] = pltpu.async_copy(
                gbuf.at[(t - 1) % 3, pl.ds(0, sizes[t - 1])],
                out.at[pl.ds(rows[t - 1], sizes[t - 1])],
                wsem.at[(t - 1) % 3])
    last = NU - 1
    gets[last].wait()
    puts[last] = pltpu.async_copy(
        gbuf.at[last % 3, pl.ds(0, sizes[last])],
        out.at[pl.ds(rows[last], sizes[last])], wsem.at[last % 3])
    puts[NU - 3].wait()
    puts[NU - 2].wait()
    puts[NU - 1].wait()

    ua = NW * A_PIPE + wid

    @pl.when(ua < A_UNITS)
    def _a_tail():
        row0 = _a_unit(props_v, idx_v, ua, 0)
        pltpu.async_copy(fm.at[idx_v.at[0]], gbuf.at[0], gsem.at[0]).wait()
        pltpu.sync_copy(gbuf.at[0], out.at[pl.ds(row0, CHUNK)])

    sb = NW + wid

    @pl.when(sb < NSEG)
    def _b_tail():
        row0 = _b_unit(props_v, idx_v, sb, 0)
        pltpu.async_copy(fm.at[idx_v.at[0, pl.ds(0, TAIL)]],
                         gbuf.at[0, pl.ds(0, TAIL)], gsem.at[0]).wait()
        pltpu.sync_copy(gbuf.at[0, pl.ds(0, TAIL)],
                        out.at[pl.ds(row0, TAIL)])


_sc_gather = functools.partial(
    pl.kernel,
    out_type=jax.ShapeDtypeStruct((TOTAL_ROWS, C), jnp.float32),
    mesh=plsc.VectorSubcoreMesh(
        core_axis_name="c", subcore_axis_name="s",
        num_cores=NC, num_subcores=NS),
    scratch_types=[
        pltpu.VMEM((4, NPAD), jnp.float32),
        pltpu.VMEM((2, CHUNK), jnp.int32),
        pltpu.VMEM((3, CHUNK, C), jnp.float32),
        pltpu.SemaphoreType.DMA((3,)),
        pltpu.SemaphoreType.DMA((3,)),
    ],
    compiler_params=pltpu.CompilerParams(needs_layout_passes=False),
)(_body)


@jax.jit
def kernel(feature_map, proposals):
    fm = feature_map.reshape(H * W, C)
    props = jnp.zeros((4, NPAD), jnp.float32).at[:, :N].set(proposals[0].T)
    out = _sc_gather(fm, props)
    return jnp.transpose(
        out.reshape(PH, PW, N, C), (2, 0, 1, 3))[None]
